# baseline (speedup 1.0000x reference)
import jax
import jax.numpy as jnp
from jax import lax
from jax.experimental import pallas as pl
from jax.experimental.pallas import tpu as pltpu
from jax.experimental.pallas import tpu_sc as plsc

VOCAB = 1000000
EMBED_DIM = 32
BATCH = 4096
HIST_LEN = 200

NC = 2
NS = 16
NW = NC * NS

ROWS_PER_STREAM = 128
TOTAL_ROWS = BATCH * HIST_LEN
ROWS_PER_W = TOTAL_ROWS // NW
CHUNKS = ROWS_PER_W // ROWS_PER_STREAM
GROUP = 8
GROUPS = CHUNKS // GROUP


def _make_gather():
    mesh = plsc.VectorSubcoreMesh(core_axis_name="c", subcore_axis_name="s")

    def body(idx_hbm, table_hbm, out_hbm, idx_v, rows_v, sem_g, sem_o):
        wid = lax.axis_index("s") * NC + lax.axis_index("c")
        pltpu.sync_copy(idx_hbm.at[wid], idx_v)

        def out_desc(g, slot):
            return pltpu.make_async_copy(
                rows_v.at[slot],
                out_hbm.at[wid, g],
                sem_o,
            )

        def group(g, carry):
            slot = lax.rem(g, 2)
            base = g * GROUP

            @pl.when(g >= 2)
            def _():
                out_desc(g - 2, slot).wait()

            pltpu.async_copy(
                table_hbm.at[idx_v.at[pl.ds(base * ROWS_PER_STREAM,
                                            GROUP * ROWS_PER_STREAM)]],
                rows_v.at[slot],
                sem_g,
            ).wait()
            out_desc(g, slot).start()
            return carry

        lax.fori_loop(0, GROUPS, group, 0)
        for g in (GROUPS - 2, GROUPS - 1):
            out_desc(g, g % 2).wait()

    kern = pl.kernel(
        body,
        out_type=jax.ShapeDtypeStruct(
            (NW, GROUPS, GROUP * ROWS_PER_STREAM, EMBED_DIM), jnp.float32),
        mesh=mesh,
        scratch_types=[
            pltpu.VMEM((ROWS_PER_W,), jnp.int32),
            pltpu.VMEM((2, GROUP * ROWS_PER_STREAM, EMBED_DIM), jnp.float32),
            pltpu.SemaphoreType.DMA,
            pltpu.SemaphoreType.DMA,
        ],
        compiler_params=pltpu.CompilerParams(use_tc_tiling_on_sc=False),
    )
    return kern


_gather = _make_gather()


def kernel(inputs, embedding_matrix):
    idx = inputs.astype(jnp.int32).reshape(NW, ROWS_PER_W)
    out = _gather(idx, embedding_matrix)
    return out.reshape(BATCH, HIST_LEN, EMBED_DIM)


# speedup vs baseline: 1.5409x; 1.5409x over previous
import jax
import jax.numpy as jnp
from jax import lax
from jax.experimental import pallas as pl
from jax.experimental.pallas import tpu as pltpu
from jax.experimental.pallas import tpu_sc as plsc

VOCAB = 1000000
EMBED_DIM = 32
BATCH = 4096
HIST_LEN = 200

NC = 2
NS = 16
NW = NC * NS
LANES = 16

GRAN = 16
NGRAN = VOCAB * EMBED_DIM // GRAN
BB = BATCH // NW
GH = 4
GROUPS = HIST_LEN // GH
GIDX = GH * BB * 2
PITCH = 129


def _make_gather():
    mesh = plsc.VectorSubcoreMesh(core_axis_name="c", subcore_axis_name="s")

    def body(idx_hbm, table_hbm, out_hbm, idx_v, idx2_v, rows_v, rowst_v,
             sem_g, sem_o):
        wid = lax.axis_index("s") * NC + lax.axis_index("c")
        pltpu.sync_copy(idx_hbm.at[:, pl.ds(wid * BB, BB)], idx_v)

        lane = lax.iota(jnp.int32, LANES)
        dt_lo = lax.shift_right_logical(lane, 3)
        dt_hi = dt_lo + 2
        dr = lax.rem(lane, 8)

        def gather_desc(slot):
            return pltpu.make_async_copy(
                table_hbm.at[idx2_v.at[slot]], rows_v.at[slot], sem_g)

        def out_desc(g, hh, slot):
            return pltpu.make_async_copy(
                rowst_v.at[slot, hh, :, :, pl.ds(0, BB)],
                out_hbm.at[g * GH + hh, :, wid],
                sem_o,
            )

        def expand(g, slot):
            dst = idx2_v.at[slot]
            for hh in range(GH):
                h = g * GH + hh
                for c in range(BB // LANES):
                    v = idx_v[h, pl.ds(c * LANES, LANES)]
                    v2 = v + v
                    pos = (hh * 2 * BB + 2 * c * LANES) + 2 * lane
                    plsc.store_scatter(dst, [pos], v2)
                    plsc.store_scatter(dst, [pos + 1], v2 + 1)

        def transpose(g, slot):
            for hh in range(GH):
                tref = rowst_v.at[slot, hh]
                base = hh * 2 * BB

                def tbody(b, carry):
                    bvec = lane * 0 + b
                    v0 = rows_v[slot, base + 2 * b]
                    v1 = rows_v[slot, base + 2 * b + 1]
                    plsc.store_scatter(tref, [dt_lo, dr, bvec], v0)
                    plsc.store_scatter(tref, [dt_hi, dr, bvec], v1)
                    return carry

                lax.fori_loop(0, BB, tbody, 0, unroll=8)

        expand(0, 0)
        gather_desc(0).start()

        def group(g, carry):
            slot = lax.rem(g, 2)

            @pl.when(g + 1 < GROUPS)
            def _():
                nslot = lax.rem(g + 1, 2)
                expand(g + 1, nslot)
                gather_desc(nslot).start()

            gather_desc(slot).wait()

            @pl.when(g >= 2)
            def _():
                for hh in range(GH):
                    out_desc(g - 2, hh, slot).wait()

            transpose(g, slot)
            for hh in range(GH):
                out_desc(g, hh, slot).start()
            return carry

        lax.fori_loop(0, GROUPS, group, 0)
        for g in (GROUPS - 2, GROUPS - 1):
            for hh in range(GH):
                out_desc(g, hh, g % 2).wait()

    kern = pl.kernel(
        body,
        out_type=jax.ShapeDtypeStruct(
            (HIST_LEN, EMBED_DIM // 8, NW, 8, BB), jnp.float32),
        mesh=mesh,
        scratch_types=[
            pltpu.VMEM((HIST_LEN, BB), jnp.int32),
            pltpu.VMEM((2, GIDX), jnp.int32),
            pltpu.VMEM((2, GIDX, GRAN), jnp.float32),
            pltpu.VMEM((2, GH, EMBED_DIM // 8, 8, PITCH), jnp.float32),
            pltpu.SemaphoreType.DMA,
            pltpu.SemaphoreType.DMA,
        ],
        compiler_params=pltpu.CompilerParams(
            use_tc_tiling_on_sc=False, needs_layout_passes=False),
    )
    return kern


_gather = _make_gather()


def kernel(inputs, embedding_matrix):
    idx = inputs.astype(jnp.int32).T
    table = embedding_matrix.reshape(NGRAN, GRAN)
    out5 = _gather(idx, table)
    return out5.transpose(2, 4, 0, 1, 3).reshape(BATCH, HIST_LEN, EMBED_DIM)


# speedup vs baseline: 1.5706x; 1.0193x over previous
import jax
import jax.numpy as jnp
from jax import lax
from jax.experimental import pallas as pl
from jax.experimental.pallas import tpu as pltpu
from jax.experimental.pallas import tpu_sc as plsc

VOCAB = 1000000
EMBED_DIM = 32
BATCH = 4096
HIST_LEN = 200

NC = 2
NS = 16
NW = NC * NS
LANES = 16

GRAN = 16
NGRAN = VOCAB * EMBED_DIM // GRAN
BB = BATCH // NW
GH = 4
GROUPS = HIST_LEN // GH
GIDX = GH * BB * 2
PITCH = 129


def _make_gather():
    mesh = plsc.VectorSubcoreMesh(core_axis_name="c", subcore_axis_name="s")

    def body(idx_hbm, table_hbm, out_hbm, idx_v, idx2_v, rows_v, rowst_v,
             sem_g, sem_o):
        wid = lax.axis_index("s") * NC + lax.axis_index("c")
        pltpu.sync_copy(idx_hbm.at[:, pl.ds(wid * BB, BB)], idx_v)

        lane = lax.iota(jnp.int32, LANES)
        dt_lo = lax.shift_right_logical(lane, 3)
        dt_hi = dt_lo + 2
        dr = lax.rem(lane, 8)

        def gather_desc(slot):
            return pltpu.make_async_copy(
                table_hbm.at[idx2_v.at[slot]], rows_v.at[slot], sem_g)

        def out_desc(g, hh, slot):
            return pltpu.make_async_copy(
                rowst_v.at[slot, hh, :, :, pl.ds(0, BB)],
                out_hbm.at[g * GH + hh, :, wid],
                sem_o,
            )

        def expand(g, slot):
            dst = idx2_v.at[slot]
            for hh in range(GH):
                h = g * GH + hh
                for c in range(BB // LANES):
                    v = idx_v[h, pl.ds(c * LANES, LANES)]
                    v2 = v * 8
                    pos = (hh * 2 * BB + 2 * c * LANES) + 2 * lane
                    plsc.store_scatter(dst, [pos], v2)
                    plsc.store_scatter(dst, [pos + 1], v2 + 1)

        def transpose(g, slot):
            for hh in range(GH):
                tref = rowst_v.at[slot, hh]
                base = hh * 2 * BB

                def tbody(b, carry):
                    bvec = lane * 0 + b
                    v0 = rows_v[slot, base + 2 * b]
                    v1 = rows_v[slot, base + 2 * b + 1]
                    plsc.store_scatter(tref, [dt_lo, dr, bvec], v0)
                    plsc.store_scatter(tref, [dt_hi, dr, bvec], v1)
                    return carry

                lax.fori_loop(0, BB, tbody, 0, unroll=8)

        expand(0, 0)
        gather_desc(0).start()

        def group(g, carry):
            slot = lax.rem(g, 2)

            @pl.when(g + 1 < GROUPS)
            def _():
                nslot = lax.rem(g + 1, 2)
                expand(g + 1, nslot)
                gather_desc(nslot).start()

            gather_desc(slot).wait()

            @pl.when(g >= 2)
            def _():
                for hh in range(GH):
                    out_desc(g - 2, hh, slot).wait()

            transpose(g, slot)
            for hh in range(GH):
                out_desc(g, hh, slot).start()
            return carry

        lax.fori_loop(0, GROUPS, group, 0)
        for g in (GROUPS - 2, GROUPS - 1):
            for hh in range(GH):
                out_desc(g, hh, g % 2).wait()

    kern = pl.kernel(
        body,
        out_type=jax.ShapeDtypeStruct(
            (HIST_LEN, EMBED_DIM // 8, NW, 8, BB), jnp.float32),
        mesh=mesh,
        scratch_types=[
            pltpu.VMEM((HIST_LEN, BB), jnp.int32),
            pltpu.VMEM((2, GIDX), jnp.int32),
            pltpu.VMEM((2, GIDX, GRAN), jnp.float32),
            pltpu.VMEM((2, GH, EMBED_DIM // 8, 8, PITCH), jnp.float32),
            pltpu.SemaphoreType.DMA,
            pltpu.SemaphoreType.DMA,
        ],
        compiler_params=pltpu.CompilerParams(
            use_tc_tiling_on_sc=False, needs_layout_passes=False),
    )
    return kern


_gather = _make_gather()


def kernel(inputs, embedding_matrix):
    idx = inputs.astype(jnp.int32).T
    table = jnp.pad(embedding_matrix, ((0, 0), (0, 96))).reshape(8000000, GRAN)
    out5 = _gather(idx, table)
    return out5.transpose(2, 4, 0, 1, 3).reshape(BATCH, HIST_LEN, EMBED_DIM)
